# baseline (speedup 1.0000x reference)
import functools

import jax
import jax.numpy as jnp
from jax import lax
from jax.experimental import pallas as pl
from jax.experimental.pallas import tpu as pltpu
from jax.experimental.pallas import tpu_sc as plsc

_SMOOTH = 0.1


def _loss_body(x_ref, out_ref, m_ref, s_ref, sum_ref, *, nj, cb, c, rb, nrows):
    i = pl.program_id(0)
    j = pl.program_id(1)

    @pl.when(j == 0)
    def _init_row_state():
        m_ref[...] = jnp.full((rb, 1), -jnp.inf, dtype=jnp.float32)
        s_ref[...] = jnp.zeros((rb, 1), dtype=jnp.float32)
        sum_ref[...] = jnp.zeros((rb, 1), dtype=jnp.float32)

    @pl.when((i == 0) & (j == 0))
    def _init_out():
        out_ref[0, 0] = 0.0

    x = x_ref[...]

    def _update(xv, xs):
        sum_ref[...] += jnp.sum(xs, axis=1, keepdims=True)
        m_old = m_ref[...]
        m_new = jnp.maximum(m_old, jnp.max(xv, axis=1, keepdims=True))
        e = jnp.exp(xv - m_new)
        s_ref[...] = (s_ref[...] * jnp.exp(m_old - m_new)
                      + jnp.sum(e, axis=1, keepdims=True))
        m_ref[...] = m_new

    @pl.when(j < nj - 1)
    def _full_block():
        _update(x, x)

    @pl.when(j == nj - 1)
    def _tail_block():
        col = lax.broadcasted_iota(jnp.int32, (rb, cb), 1)
        valid = col < (c - (nj - 1) * cb)
        _update(jnp.where(valid, x, -jnp.inf), jnp.where(valid, x, 0.0))

        a = _SMOOTH / (c - 1)
        b = (1.0 - _SMOOTH) - a
        lse = m_ref[...] + jnp.log(s_ref[...])
        row_loss = a * (c * lse - sum_ref[...]) + b * lse
        out_ref[0, 0] += jnp.sum(row_loss) / nrows


def _tc_loss(pred):
    nrows, c = pred.shape
    rb = 256 if nrows % 256 == 0 else nrows
    cb = min(8192, ((c + 127) // 128) * 128)
    ni = nrows // rb
    nj = (c + cb - 1) // cb

    out = pl.pallas_call(
        functools.partial(_loss_body, nj=nj, cb=cb, c=c, rb=rb, nrows=nrows),
        grid=(ni, nj),
        in_specs=[pl.BlockSpec((rb, cb), lambda i, j: (i, j))],
        out_specs=pl.BlockSpec(memory_space=pltpu.SMEM),
        out_shape=jax.ShapeDtypeStruct((1, 1), jnp.float32),
        scratch_shapes=[
            pltpu.VMEM((rb, 1), jnp.float32),
            pltpu.VMEM((rb, 1), jnp.float32),
            pltpu.VMEM((rb, 1), jnp.float32),
        ],
        compiler_params=pltpu.CompilerParams(
            dimension_semantics=("arbitrary", "arbitrary"),
        ),
    )(pred)
    return out.reshape(())


def _sc_gather_partials(pred_flat, target, *, nrows, c, nw, per_w):
    nchunk = per_w // 16

    @functools.partial(
        pl.kernel,
        out_type=jax.ShapeDtypeStruct((nw, 16), jnp.float32),
        mesh=plsc.VectorSubcoreMesh(core_axis_name="c", subcore_axis_name="s"),
        scratch_types=[
            pltpu.VMEM((per_w,), jnp.int32),
            pltpu.VMEM((per_w,), jnp.int32),
            pltpu.VMEM((per_w,), jnp.float32),
            pltpu.VMEM((16,), jnp.float32),
            pltpu.SemaphoreType.DMA,
        ],
    )
    def k(t_hbm, flat_hbm, out_hbm, t_v, idx_v, p_v, acc_v, sem):
        nc = 2
        wid = lax.axis_index("s") * nc + lax.axis_index("c")
        base = wid * per_w
        pltpu.sync_copy(t_hbm.at[pl.ds(base, per_w)], t_v)
        lane = lax.broadcasted_iota(jnp.int32, (16,), 0)
        for kk in range(nchunk):
            rowid = base + kk * 16 + lane
            idx_v[pl.ds(kk * 16, 16)] = t_v[pl.ds(kk * 16, 16)] + rowid * c
        pltpu.async_copy(flat_hbm.at[idx_v], p_v, sem).wait()
        acc = jnp.zeros((16,), jnp.float32)
        for kk in range(nchunk):
            acc = acc + p_v[pl.ds(kk * 16, 16)]
        acc_v[...] = acc
        pltpu.sync_copy(acc_v, out_hbm.at[wid])

    return k(target, pred_flat)


def kernel(pred, target):
    nrows, c = pred.shape
    nw = 32
    per_w = nrows // nw

    tc_part = _tc_loss(pred)
    p_parts = _sc_gather_partials(
        pred.reshape(-1), target.astype(jnp.int32),
        nrows=nrows, c=c, nw=nw, per_w=per_w)

    a = _SMOOTH / (c - 1)
    b = (1.0 - _SMOOTH) - a
    return (tc_part - b * jnp.sum(p_parts) / nrows).reshape(())


# speedup vs baseline: 1.9510x; 1.9510x over previous
import functools

import jax
import jax.numpy as jnp
from jax import lax
from jax.experimental import pallas as pl
from jax.experimental.pallas import tpu as pltpu

_SMOOTH = 0.1
_GK = 16


def _loss_body(x_ref, out_ref, m_ref, s_ref, sum_ref, *, nj, cb, c, rb, nrows):
    i = pl.program_id(0)
    j = pl.program_id(1)

    @pl.when(j == 0)
    def _init_row_state():
        m_ref[...] = jnp.full((rb, 1), -jnp.inf, dtype=jnp.float32)
        s_ref[...] = jnp.zeros((rb, 1), dtype=jnp.float32)
        sum_ref[...] = jnp.zeros((rb, 1), dtype=jnp.float32)

    @pl.when((i == 0) & (j == 0))
    def _init_out():
        out_ref[0, 0] = 0.0

    x = x_ref[...]

    def _update(xv, xs):
        sum_ref[...] += jnp.sum(xs, axis=1, keepdims=True)
        m_old = m_ref[...]
        m_new = jnp.maximum(m_old, jnp.max(xv, axis=1, keepdims=True))
        e = jnp.exp(xv - m_new)
        s_ref[...] = (s_ref[...] * jnp.exp(m_old - m_new)
                      + jnp.sum(e, axis=1, keepdims=True))
        m_ref[...] = m_new

    @pl.when(j < nj - 1)
    def _full_block():
        _update(x, x)

    @pl.when(j == nj - 1)
    def _tail_block():
        col = lax.broadcasted_iota(jnp.int32, (rb, cb), 1)
        valid = col < (c - (nj - 1) * cb)
        _update(jnp.where(valid, x, -jnp.inf), jnp.where(valid, x, 0.0))

        a = _SMOOTH / (c - 1)
        b = (1.0 - _SMOOTH) - a
        lse = m_ref[...] + jnp.log(s_ref[...])
        row_loss = a * (c * lse - sum_ref[...]) + b * lse
        out_ref[0, 0] += jnp.sum(row_loss) / nrows


def _tc_loss(pred):
    nrows, c = pred.shape
    rb = 256 if nrows % 256 == 0 else nrows
    cb = min(8192, ((c + 127) // 128) * 128)
    ni = nrows // rb
    nj = (c + cb - 1) // cb

    out = pl.pallas_call(
        functools.partial(_loss_body, nj=nj, cb=cb, c=c, rb=rb, nrows=nrows),
        grid=(ni, nj),
        in_specs=[pl.BlockSpec((rb, cb), lambda i, j: (i, j))],
        out_specs=pl.BlockSpec(memory_space=pltpu.SMEM),
        out_shape=jax.ShapeDtypeStruct((1, 1), jnp.float32),
        scratch_shapes=[
            pltpu.VMEM((rb, 1), jnp.float32),
            pltpu.VMEM((rb, 1), jnp.float32),
            pltpu.VMEM((rb, 1), jnp.float32),
        ],
        compiler_params=pltpu.CompilerParams(
            dimension_semantics=("arbitrary", "arbitrary"),
        ),
    )(pred)
    return out.reshape(())


def _gather_body(t_smem, *refs):
    xs, out_ref = refs[:-1], refs[-1]
    g = pl.program_id(0)

    @pl.when(g == 0)
    def _init():
        out_ref[0, 0] = 0.0

    lane = lax.broadcasted_iota(jnp.int32, (1, 128), 1)
    acc = jnp.zeros((1, 128), jnp.float32)
    for k in range(_GK):
        t_lane = t_smem[g * _GK + k] % 128
        row = xs[k][k % 8:k % 8 + 1, :]
        acc = acc + jnp.where(lane == t_lane, row, 0.0)
    out_ref[0, 0] += jnp.sum(acc)


def _target_sum(pred, target):
    nrows, _ = pred.shape
    grid = nrows // _GK

    def _mk_index_map(k):
        return lambda g, t: (g * (_GK // 8) + k // 8, t[g * _GK + k] // 128)

    out = pl.pallas_call(
        _gather_body,
        grid_spec=pltpu.PrefetchScalarGridSpec(
            num_scalar_prefetch=1,
            grid=(grid,),
            in_specs=[pl.BlockSpec((8, 128), _mk_index_map(k))
                      for k in range(_GK)],
            out_specs=pl.BlockSpec(memory_space=pltpu.SMEM),
        ),
        out_shape=jax.ShapeDtypeStruct((1, 1), jnp.float32),
        compiler_params=pltpu.CompilerParams(
            dimension_semantics=("arbitrary",),
        ),
    )(target.astype(jnp.int32), *([pred] * _GK))
    return out.reshape(())


def kernel(pred, target):
    nrows, c = pred.shape
    tc_part = _tc_loss(pred)
    p_sum = _target_sum(pred, target)
    a = _SMOOTH / (c - 1)
    b = (1.0 - _SMOOTH) - a
    return (tc_part - b * p_sum / nrows).reshape(())
